# Initial kernel scaffold; baseline (speedup 1.0000x reference)
import functools

import jax
import jax.numpy as jnp
from jax import lax
from jax.experimental import pallas as pl
from jax.experimental.pallas import tpu as pltpu
from jax.experimental.pallas import tpu_sc as plsc

NB_CODE = 8192
CODE_DIM = 64
B, N, Q = 32, 2048, 4
R = B * N * Q

NC, NS = 2, 16
NW = NC * NS
RPW = R // NW
C = 128
K = 4
GROUP = C * K
G = RPW // GROUP
IDX_ROWS = RPW // C


def _vq_gather(table, idx2d):
    mesh = plsc.VectorSubcoreMesh(core_axis_name="c", subcore_axis_name="s")

    @functools.partial(
        pl.kernel,
        mesh=mesh,
        out_type=jax.ShapeDtypeStruct((R, CODE_DIM), jnp.float32),
        scratch_types=[
            pltpu.VMEM((IDX_ROWS, C), jnp.int32),
            pltpu.VMEM((GROUP, CODE_DIM), jnp.float32),
            pltpu.SemaphoreType.DMA,
        ],
    )
    def k(table_hbm, idx_hbm, out_hbm, idx_v, rows, gsem):
        wid = lax.axis_index("s") * NC + lax.axis_index("c")
        pltpu.sync_copy(idx_hbm.at[pl.ds(wid * IDX_ROWS, IDX_ROWS)], idx_v)

        def group(g, carry):
            cps = []
            for kk in range(K):
                cps.append(pltpu.async_copy(
                    table_hbm.at[idx_v.at[g * K + kk]],
                    rows.at[pl.ds(kk * C, C)],
                    gsem,
                ))
            for cp in cps:
                cp.wait()
            pltpu.sync_copy(rows,
                            out_hbm.at[pl.ds(wid * RPW + g * GROUP, GROUP)])
            return carry

        lax.fori_loop(0, G, group, 0)

    return k(table, idx2d)


def kernel(idx, codebook, padding_embedding):
    table = codebook.at[0].set(padding_embedding.reshape(CODE_DIM))
    idx2d = idx.reshape(R // C, C)
    out = _vq_gather(table, idx2d)
    return out.reshape(B, N, Q, CODE_DIM)


# speedup vs baseline: 3.3945x; 3.3945x over previous
import functools

import jax
import jax.numpy as jnp
from jax import lax
from jax.experimental import pallas as pl
from jax.experimental.pallas import tpu as pltpu
from jax.experimental.pallas import tpu_sc as plsc

NB_CODE = 8192
CODE_DIM = 64
B, N, Q = 32, 2048, 4
R = B * N * Q

NC, NS = 2, 16
NW = NC * NS
RPW = R // NW
C = 128
K = 4
GROUP = C * K
G = RPW // GROUP
IDX_ROWS = RPW // C


def _vq_gather(table, idx2d):
    mesh = plsc.VectorSubcoreMesh(core_axis_name="c", subcore_axis_name="s")

    @functools.partial(
        pl.kernel,
        mesh=mesh,
        out_type=jax.ShapeDtypeStruct((R, CODE_DIM), jnp.float32),
        compiler_params=pltpu.CompilerParams(use_tc_tiling_on_sc=False),
        scratch_types=[
            pltpu.VMEM((IDX_ROWS, C), jnp.int32),
            pltpu.VMEM((GROUP, CODE_DIM), jnp.float32),
            pltpu.SemaphoreType.DMA,
        ],
    )
    def k(table_hbm, idx_hbm, out_hbm, idx_v, rows, gsem):
        wid = lax.axis_index("s") * NC + lax.axis_index("c")
        pltpu.sync_copy(idx_hbm.at[pl.ds(wid * IDX_ROWS, IDX_ROWS)], idx_v)

        def group(g, carry):
            cps = []
            for kk in range(K):
                cps.append(pltpu.async_copy(
                    table_hbm.at[idx_v.at[g * K + kk]],
                    rows.at[pl.ds(kk * C, C)],
                    gsem,
                ))
            for cp in cps:
                cp.wait()
            pltpu.sync_copy(rows,
                            out_hbm.at[pl.ds(wid * RPW + g * GROUP, GROUP)])
            return carry

        lax.fori_loop(0, G, group, 0)

    return k(table, idx2d)


def kernel(idx, codebook, padding_embedding):
    table = codebook.at[0].set(padding_embedding.reshape(CODE_DIM))
    idx2d = idx.reshape(R // C, C)
    out = _vq_gather(table, idx2d)
    return out.reshape(B, N, Q, CODE_DIM)


# speedup vs baseline: 3.4526x; 1.0171x over previous
import functools

import jax
import jax.numpy as jnp
from jax import lax
from jax.experimental import pallas as pl
from jax.experimental.pallas import tpu as pltpu
from jax.experimental.pallas import tpu_sc as plsc

NB_CODE = 8192
CODE_DIM = 64
B, N, Q = 32, 2048, 4
R = B * N * Q

NC, NS = 2, 16
NW = NC * NS
RPW = R // NW
C = 128
K = 4
GROUP = C * K
G = RPW // GROUP
IDX_ROWS = RPW // C


def _vq_gather(table, idx2d):
    mesh = plsc.VectorSubcoreMesh(core_axis_name="c", subcore_axis_name="s")

    @functools.partial(
        pl.kernel,
        mesh=mesh,
        out_type=jax.ShapeDtypeStruct((R, CODE_DIM), jnp.float32),
        compiler_params=pltpu.CompilerParams(use_tc_tiling_on_sc=False),
        scratch_types=[
            pltpu.VMEM((IDX_ROWS, C), jnp.int32),
            pltpu.VMEM((GROUP, CODE_DIM), jnp.float32),
            pltpu.VMEM((GROUP, CODE_DIM), jnp.float32),
            pltpu.SemaphoreType.DMA,
            pltpu.SemaphoreType.DMA,
            pltpu.SemaphoreType.DMA,
            pltpu.SemaphoreType.DMA,
        ],
    )
    def k(table_hbm, idx_hbm, out_hbm, idx_v, rows_a, rows_b,
          gs_a, gs_b, os_a, os_b):
        wid = lax.axis_index("s") * NC + lax.axis_index("c")
        pltpu.sync_copy(idx_hbm.at[pl.ds(wid * IDX_ROWS, IDX_ROWS)], idx_v)
        out_base = wid * RPW

        def fire_gathers(g, rows, gsem):
            for kk in range(K):
                pltpu.async_copy(table_hbm.at[idx_v.at[g * K + kk]],
                                 rows.at[pl.ds(kk * C, C)], gsem)

        def drain_gathers(g, rows, gsem):
            for kk in range(K):
                pltpu.make_async_copy(table_hbm.at[idx_v.at[g * K + kk]],
                                      rows.at[pl.ds(kk * C, C)], gsem).wait()

        def fire_out(g, rows, osem):
            pltpu.async_copy(rows, out_hbm.at[pl.ds(out_base + g * GROUP,
                                                    GROUP)], osem)

        def wait_out(g, rows, osem):
            pltpu.make_async_copy(rows, out_hbm.at[pl.ds(out_base + g * GROUP,
                                                         GROUP)], osem).wait()

        fire_gathers(0, rows_a, gs_a)

        def body(i, carry):
            g0 = 2 * i
            g1 = g0 + 1
            drain_gathers(g0, rows_a, gs_a)

            @pl.when(i > 0)
            def _():
                wait_out(g1 - 2, rows_b, os_b)
            fire_gathers(g1, rows_b, gs_b)
            fire_out(g0, rows_a, os_a)
            drain_gathers(g1, rows_b, gs_b)

            @pl.when(i < G // 2 - 1)
            def _():
                wait_out(g0, rows_a, os_a)
                fire_gathers(g0 + 2, rows_a, gs_a)
            fire_out(g1, rows_b, os_b)
            return carry

        lax.fori_loop(0, G // 2, body, 0)
        wait_out(G - 2, rows_a, os_a)
        wait_out(G - 1, rows_b, os_b)

    return k(table, idx2d)


def kernel(idx, codebook, padding_embedding):
    table = codebook.at[0].set(padding_embedding.reshape(CODE_DIM))
    idx2d = idx.reshape(R // C, C)
    out = _vq_gather(table, idx2d)
    return out.reshape(B, N, Q, CODE_DIM)
